# Initial kernel scaffold; baseline (speedup 1.0000x reference)
import jax
import jax.numpy as jnp
from jax.experimental import pallas as pl


def kernel(codebook, idx):
    raise NotImplementedError("write your pallas kernel here")


# speedup vs baseline: 1.4602x; 1.4602x over previous
import functools

import jax
import jax.numpy as jnp
from jax import lax
from jax.experimental import pallas as pl
from jax.experimental.pallas import tpu as pltpu
from jax.experimental.pallas import tpu_sc as plsc

N_CLASSES = 256
TOTAL_BINS = 8
LANES = 16

NUM_CORES = 2
NUM_SUBCORES = 16
NUM_WORKERS = NUM_CORES * NUM_SUBCORES

CHUNK = 6400


def _make_sc_lookup(n_idx: int):
    assert n_idx % NUM_WORKERS == 0
    per_w = n_idx // NUM_WORKERS
    assert per_w % CHUNK == 0
    n_chunks = per_w // CHUNK

    mesh = plsc.VectorSubcoreMesh(
        core_axis_name="c", subcore_axis_name="s",
        num_cores=NUM_CORES, num_subcores=NUM_SUBCORES)

    @functools.partial(
        pl.kernel,
        out_type=jax.ShapeDtypeStruct((n_idx * TOTAL_BINS,), jnp.float32),
        mesh=mesh,
        scratch_types=[
            pltpu.VMEM((N_CLASSES * TOTAL_BINS,), jnp.float32),
            pltpu.VMEM((CHUNK,), jnp.int32),
            pltpu.VMEM((CHUNK * TOTAL_BINS,), jnp.float32),
        ],
        compiler_params=pltpu.CompilerParams(needs_layout_passes=False),
    )
    def lookup(table_hbm, idx_hbm, out_hbm, table_v, idx_v, out_v):
        wid = lax.axis_index("s") * NUM_CORES + lax.axis_index("c")
        base = wid * per_w
        pltpu.sync_copy(table_hbm, table_v)
        lane = lax.iota(jnp.int32, LANES)

        def chunk_body(ci, carry):
            off = base + ci * CHUNK
            pltpu.sync_copy(idx_hbm.at[pl.ds(off, CHUNK)], idx_v)

            def vec_body(vi, c2):
                idxv = idx_v[pl.ds(vi * LANES, LANES)]
                srcs = idxv * TOTAL_BINS
                dsts = (vi * LANES + lane) * TOTAL_BINS
                for j in range(TOTAL_BINS):
                    col = plsc.load_gather(table_v, [srcs + j])
                    plsc.store_scatter(out_v, [dsts + j], col)
                return c2

            lax.fori_loop(0, CHUNK // LANES, vec_body, 0)
            pltpu.sync_copy(
                out_v, out_hbm.at[pl.ds(off * TOTAL_BINS, CHUNK * TOTAL_BINS)])
            return carry

        lax.fori_loop(0, n_chunks, chunk_body, 0)

    return lookup


def kernel(codebook, idx):
    n_idx = idx.size
    flat = _make_sc_lookup(n_idx)(codebook.reshape(-1), idx.reshape(-1))
    return flat.reshape(idx.shape + (2, TOTAL_BINS // 2))


# speedup vs baseline: 3.2439x; 2.2215x over previous
import functools

import jax
import jax.numpy as jnp
from jax import lax
from jax.experimental import pallas as pl
from jax.experimental.pallas import tpu as pltpu
from jax.experimental.pallas import tpu_sc as plsc

N_CLASSES = 256
TOTAL_BINS = 8

NUM_CORES = 2
NUM_SUBCORES = 16
NUM_WORKERS = NUM_CORES * NUM_SUBCORES

IDX_MINOR = 128
ROWS = 16
CHUNK = ROWS * IDX_MINOR


def _make_sc_lookup(n_idx: int):
    assert n_idx % (NUM_WORKERS * CHUNK) == 0
    per_w = n_idx // NUM_WORKERS
    n_chunks = per_w // CHUNK

    mesh = plsc.VectorSubcoreMesh(
        core_axis_name="c", subcore_axis_name="s",
        num_cores=NUM_CORES, num_subcores=NUM_SUBCORES)

    @functools.partial(
        pl.kernel,
        out_type=jax.ShapeDtypeStruct((n_idx, TOTAL_BINS), jnp.float32),
        mesh=mesh,
        scratch_types=[
            pltpu.VMEM((ROWS, IDX_MINOR), jnp.int32),
            pltpu.VMEM((CHUNK, TOTAL_BINS), jnp.float32),
            pltpu.SemaphoreType.DMA,
        ],
        compiler_params=pltpu.CompilerParams(use_tc_tiling_on_sc=False),
    )
    def lookup(table_hbm, idx_hbm, out_hbm, idx_v, rows_v, sem):
        wid = lax.axis_index("s") * NUM_CORES + lax.axis_index("c")
        base_row = wid * (per_w // IDX_MINOR)

        def chunk_body(ci, carry):
            row0 = base_row + ci * ROWS
            pltpu.sync_copy(idx_hbm.at[pl.ds(row0, ROWS)], idx_v)
            copies = []
            for b in range(ROWS):
                copies.append(pltpu.async_copy(
                    table_hbm.at[idx_v.at[b]],
                    rows_v.at[pl.ds(b * IDX_MINOR, IDX_MINOR)],
                    sem))
            for c in copies:
                c.wait()
            off = (base_row + ci * ROWS) * IDX_MINOR
            pltpu.sync_copy(rows_v, out_hbm.at[pl.ds(off, CHUNK)])
            return carry

        lax.fori_loop(0, n_chunks, chunk_body, 0)

    return lookup


def kernel(codebook, idx):
    n_idx = idx.size
    idx2d = idx.reshape(n_idx // IDX_MINOR, IDX_MINOR)
    rows = _make_sc_lookup(n_idx)(codebook, idx2d)
    return rows.reshape(idx.shape + (2, TOTAL_BINS // 2))
